# Initial kernel scaffold; baseline (speedup 1.0000x reference)
import jax
import jax.numpy as jnp
from jax.experimental import pallas as pl


def kernel(mem, indices, values):
    raise NotImplementedError("write your pallas kernel here")


# speedup vs baseline: 13.9411x; 13.9411x over previous
import jax
import jax.numpy as jnp
from jax import lax
from jax.experimental import pallas as pl
from jax.experimental.pallas import tpu as pltpu
from jax.experimental.pallas import tpu_sc as plsc

_NC = 2
_NT = 16
_L = 16


def kernel(mem, indices, values):
    M, D = mem.shape
    B = values.shape[0]
    PS = (M // _NT + 7) // 8 * 8
    SLAB = _NT * PS
    OCHUNK = B // (_NC * _NT)
    OR = OCHUNK // 128
    SCAN = B // _L
    mesh = plsc.VectorSubcoreMesh(core_axis_name="c", subcore_axis_name="s")
    i32 = jnp.int32

    def body(idx_hbm, val_hbm, out_hbm,
             idxb, tloc, oidx, ow, rows, t_hbm, sem):
        cid = lax.axis_index("c")
        sid = lax.axis_index("s")
        obase = cid * (B // _NC) + sid * OCHUNK
        lane = lax.iota(jnp.int32, _L)
        lo = sid * PS

        for r in range(B // 128):
            pltpu.sync_copy(idx_hbm.at[pl.ds(r * 128, 128)], idxb.at[r])

        def scan(i, _):
            r = i // (128 // _L)
            c = i % (128 // _L)
            slot = idxb[r, pl.ds(c * _L, _L)]
            tgt = slot - lo
            owned = (tgt >= 0) & (tgt < PS)
            tgt = jnp.minimum(jnp.maximum(tgt, 0), PS - 1)
            j = i * _L + lane
            for l in range(_L):
                plsc.store_scatter(tloc, [tgt], j, mask=owned & (lane == l))
            return 0
        lax.fori_loop(0, SCAN, scan, 0)

        pltpu.sync_copy(tloc, t_hbm.at[pl.ds(cid * SLAB + lo, PS)])
        plsc.subcore_barrier()

        for r in range(OR):
            pltpu.sync_copy(idx_hbm.at[pl.ds(obase + r * 128, 128)],
                            oidx.at[r])

        def shift(i, _):
            r = i // (128 // _L)
            c = i % (128 // _L)
            oidx[r, pl.ds(c * _L, _L)] = (oidx[r, pl.ds(c * _L, _L)]
                                          + cid * SLAB)
            return 0
        lax.fori_loop(0, OCHUNK // _L, shift, 0)

        for r in range(OR):
            pltpu.sync_copy(t_hbm.at[oidx.at[r]], ow.at[r])
        for h in range(2):
            copies = [pltpu.async_copy(val_hbm.at[ow.at[h * (OR // 2) + r]],
                                       rows.at[pl.ds(r * 128, 128)], sem)
                      for r in range(OR // 2)]
            for c in copies:
                c.wait()
            pltpu.sync_copy(
                rows, out_hbm.at[pl.ds(obase + h * (OCHUNK // 2),
                                       OCHUNK // 2)])

    run = pl.kernel(
        body,
        out_type=jax.ShapeDtypeStruct((B, D), jnp.float32),
        mesh=mesh,
        scratch_types=[
            pltpu.VMEM((B // 128, 128), i32),
            pltpu.VMEM((PS,), i32),
            pltpu.VMEM((OCHUNK // 128, 128), i32),
            pltpu.VMEM((OCHUNK // 128, 128), i32),
            pltpu.VMEM((OCHUNK // 2, D), jnp.float32),
            pltpu.MemorySpace.HBM((_NC * SLAB,), i32),
            pltpu.SemaphoreType.DMA,
        ],
        compiler_params=pltpu.CompilerParams(
            needs_layout_passes=False, use_tc_tiling_on_sc=False),
    )
    return run(indices, values)


# speedup vs baseline: 24.1554x; 1.7327x over previous
import jax
import jax.numpy as jnp
from jax import lax
from jax.experimental import pallas as pl
from jax.experimental.pallas import tpu as pltpu
from jax.experimental.pallas import tpu_sc as plsc

_NC = 2
_NT = 16
_L = 16


def kernel(mem, indices, values):
    M, D = mem.shape
    B = values.shape[0]
    PS = (M // _NT + 7) // 8 * 8
    SLAB = _NT * PS
    OCHUNK = B // (_NC * _NT)
    OR = OCHUNK // 128
    SCAN = B // _L
    mesh = plsc.VectorSubcoreMesh(core_axis_name="c", subcore_axis_name="s")
    i32 = jnp.int32

    def body(idx_hbm, val_hbm, out_hbm,
             idxb, tloc, oidx, ow, rows, t_hbm, sem):
        cid = lax.axis_index("c")
        sid = lax.axis_index("s")
        obase = cid * (B // _NC) + sid * OCHUNK
        lane = lax.iota(jnp.int32, _L)
        lo = sid * PS

        pltpu.sync_copy(idx_hbm, idxb)

        def scan(i, _):
            slot = idxb[pl.ds(i * _L, _L)]
            tgt = slot - lo
            owned = (tgt >= 0) & (tgt < PS)
            tgt = jnp.minimum(jnp.maximum(tgt, 0), PS - 1)
            j = i * _L + lane
            for l in range(_L):
                plsc.store_scatter(tloc, [tgt], j, mask=owned & (lane == l))
            return 0
        lax.fori_loop(0, SCAN, scan, 0)

        pltpu.sync_copy(tloc, t_hbm.at[pl.ds(cid * SLAB + lo, PS)])
        plsc.subcore_barrier()

        for r in range(OR):
            pltpu.sync_copy(idx_hbm.at[pl.ds(obase + r * 128, 128)],
                            oidx.at[r])

        def shift(i, _):
            r = i // (128 // _L)
            c = i % (128 // _L)
            oidx[r, pl.ds(c * _L, _L)] = (oidx[r, pl.ds(c * _L, _L)]
                                          + cid * SLAB)
            return 0
        lax.fori_loop(0, OCHUNK // _L, shift, 0)

        for r in range(OR):
            pltpu.sync_copy(t_hbm.at[oidx.at[r]], ow.at[r])
        for h in range(2):
            copies = [pltpu.async_copy(val_hbm.at[ow.at[h * (OR // 2) + r]],
                                       rows.at[pl.ds(r * 128, 128)], sem)
                      for r in range(OR // 2)]
            for c in copies:
                c.wait()
            pltpu.sync_copy(
                rows, out_hbm.at[pl.ds(obase + h * (OCHUNK // 2),
                                       OCHUNK // 2)])

    run = pl.kernel(
        body,
        out_type=jax.ShapeDtypeStruct((B, D), jnp.float32),
        mesh=mesh,
        scratch_types=[
            pltpu.VMEM((B,), i32),
            pltpu.VMEM((PS,), i32),
            pltpu.VMEM((OCHUNK // 128, 128), i32),
            pltpu.VMEM((OCHUNK // 128, 128), i32),
            pltpu.VMEM((OCHUNK // 2, D), jnp.float32),
            pltpu.MemorySpace.HBM((_NC * SLAB,), i32),
            pltpu.SemaphoreType.DMA,
        ],
        compiler_params=pltpu.CompilerParams(
            needs_layout_passes=False, use_tc_tiling_on_sc=False),
    )
    return run(indices, values)


# speedup vs baseline: 25.5368x; 1.0572x over previous
import jax
import jax.numpy as jnp
from jax import lax
from jax.experimental import pallas as pl
from jax.experimental.pallas import tpu as pltpu
from jax.experimental.pallas import tpu_sc as plsc

_NC = 2
_NT = 16
_L = 16


def kernel(mem, indices, values):
    M, D = mem.shape
    B = values.shape[0]
    PS = (M // _NT + 7) // 8 * 8
    SLAB = _NT * PS
    OCHUNK = B // (_NC * _NT)
    OR = OCHUNK // 128
    SCAN = B // _L
    mesh = plsc.VectorSubcoreMesh(core_axis_name="c", subcore_axis_name="s")
    i32 = jnp.int32

    def body(idx_hbm, val_hbm, out_hbm,
             idxb, tloc, oidx, ow, rows, t_hbm, sem):
        cid = lax.axis_index("c")
        sid = lax.axis_index("s")
        obase = cid * (B // _NC) + sid * OCHUNK
        lane = lax.iota(jnp.int32, _L)
        lo = sid * PS

        pltpu.sync_copy(idx_hbm, idxb)

        def scan(i, _):
            slot = idxb[pl.ds(i * _L, _L)]
            tgt = slot - lo
            owned = (tgt >= 0) & (tgt < PS)
            tgt = jnp.minimum(jnp.maximum(tgt, 0), PS - 1)
            j = i * _L + lane
            for l in range(_L):
                plsc.store_scatter(tloc, [tgt], j, mask=owned & (lane == l))
            return 0
        lax.fori_loop(0, SCAN, scan, 0)

        pltpu.sync_copy(tloc, t_hbm.at[pl.ds(cid * SLAB + lo, PS)])
        plsc.subcore_barrier()

        def mkoidx(i, _):
            r = i // (128 // _L)
            c = i % (128 // _L)
            oidx[r, pl.ds(c * _L, _L)] = (
                idxb[pl.ds(obase + i * _L, _L)] + cid * SLAB)
            return 0
        lax.fori_loop(0, OCHUNK // _L, mkoidx, 0)

        gcopies = [pltpu.async_copy(t_hbm.at[oidx.at[r]], ow.at[r], sem)
                   for r in range(OR)]
        for c in gcopies:
            c.wait()
        rcopies = [pltpu.async_copy(val_hbm.at[ow.at[r]],
                                    rows.at[pl.ds(r * 128, 128)], sem)
                   for r in range(OR)]
        for c in rcopies:
            c.wait()
        pltpu.sync_copy(rows, out_hbm.at[pl.ds(obase, OCHUNK)])

    run = pl.kernel(
        body,
        out_type=jax.ShapeDtypeStruct((B, D), jnp.float32),
        mesh=mesh,
        scratch_types=[
            pltpu.VMEM((B,), i32),
            pltpu.VMEM((PS,), i32),
            pltpu.VMEM((OCHUNK // 128, 128), i32),
            pltpu.VMEM((OCHUNK // 128, 128), i32),
            pltpu.VMEM((OCHUNK, D), jnp.float32),
            pltpu.MemorySpace.HBM((_NC * SLAB,), i32),
            pltpu.SemaphoreType.DMA,
        ],
        compiler_params=pltpu.CompilerParams(
            needs_layout_passes=False, use_tc_tiling_on_sc=False),
    )
    return run(indices, values)


# speedup vs baseline: 26.9650x; 1.0559x over previous
import jax
import jax.numpy as jnp
from jax import lax
from jax.experimental import pallas as pl
from jax.experimental.pallas import tpu as pltpu
from jax.experimental.pallas import tpu_sc as plsc

_NC = 2
_NT = 16
_L = 16


def kernel(mem, indices, values):
    M, D = mem.shape
    B = values.shape[0]
    PS = (M // _NT + 7) // 8 * 8
    SLAB = _NT * PS
    OCHUNK = B // (_NC * _NT)
    OR = OCHUNK // 128
    SCAN = B // _L
    mesh = plsc.VectorSubcoreMesh(core_axis_name="c", subcore_axis_name="s")
    i32 = jnp.int32

    def body(idx_hbm, val_hbm, out_hbm,
             idxb, tloc, oidx, ow, rows, t_hbm, sem):
        cid = lax.axis_index("c")
        sid = lax.axis_index("s")
        obase = cid * (B // _NC) + sid * OCHUNK
        lane = lax.iota(jnp.int32, _L)
        lo = sid * PS

        pltpu.sync_copy(idx_hbm, idxb)

        def scan(i0, _):
            for u in range(4):
                i = i0 * 4 + u
                slot = idxb[pl.ds(i * _L, _L)]
                tgt = slot - lo
                owned = (tgt >= 0) & (tgt < PS)
                tgt = jnp.minimum(jnp.maximum(tgt, 0), PS - 1)
                j = i * _L + lane
                for l in range(_L):
                    plsc.store_scatter(tloc, [tgt], j,
                                       mask=owned & (lane == l))
            return 0
        lax.fori_loop(0, SCAN // 4, scan, 0)

        pltpu.sync_copy(tloc, t_hbm.at[pl.ds(cid * SLAB + lo, PS)])
        plsc.subcore_barrier()

        def mkoidx(i, _):
            r = i // (128 // _L)
            c = i % (128 // _L)
            oidx[r, pl.ds(c * _L, _L)] = (
                idxb[pl.ds(obase + i * _L, _L)] + cid * SLAB)
            return 0
        lax.fori_loop(0, OCHUNK // _L, mkoidx, 0)

        gcopies = [pltpu.async_copy(t_hbm.at[oidx.at[r]], ow.at[r], sem)
                   for r in range(OR)]
        for c in gcopies:
            c.wait()
        rcopies = [pltpu.async_copy(val_hbm.at[ow.at[r]],
                                    rows.at[pl.ds(r * 128, 128)], sem)
                   for r in range(OR)]
        for c in rcopies:
            c.wait()
        pltpu.sync_copy(rows, out_hbm.at[pl.ds(obase, OCHUNK)])

    run = pl.kernel(
        body,
        out_type=jax.ShapeDtypeStruct((B, D), jnp.float32),
        mesh=mesh,
        scratch_types=[
            pltpu.VMEM((B,), i32),
            pltpu.VMEM((PS,), i32),
            pltpu.VMEM((OCHUNK // 128, 128), i32),
            pltpu.VMEM((OCHUNK // 128, 128), i32),
            pltpu.VMEM((OCHUNK, D), jnp.float32),
            pltpu.MemorySpace.HBM((_NC * SLAB,), i32),
            pltpu.SemaphoreType.DMA,
        ],
        compiler_params=pltpu.CompilerParams(
            needs_layout_passes=False, use_tc_tiling_on_sc=False),
    )
    return run(indices, values)
